# baseline (speedup 1.0000x reference)
import functools

import jax
import jax.numpy as jnp
from jax import lax
from jax.experimental import pallas as pl
from jax.experimental.pallas import tpu as pltpu
from jax.experimental.pallas import tpu_sc as plsc

B = 4096
D = 784
NUM_CLASSES = 10

NC = 2
NS = 16
NW = NC * NS
CHUNK = 32
NPASS = B // (NW * CHUNK)
OCN = D // 16

_LAYERS = ((0, 2, 0, True), (2, 4, 1, True), (6, 8, 2, False))
_KTOT = 14


def _gelu_exact(v):
    z = v * 0.7071067811865476
    a = jnp.abs(z)
    t = 1.0 / (1.0 + 0.3275911 * a)
    poly = t * (0.254829592 + t * (-0.284496736 + t * (1.421413741
             + t * (-1.453152027 + t * 1.061405429))))
    erf_a = 1.0 - poly * jnp.exp(-(a * a))
    erf_z = jnp.where(z < 0.0, -erf_a, erf_a)
    return 0.5 * v * (1.0 + erf_z)


def _sc_layer(src, dst, idxs, ws, bsv, k0, kk, brow, do_gelu):
    def oc_body(oc, carry):
        col = oc * 16
        bias = bsv[pl.ds(brow * D + col, 16)]
        taps = [(idxs[pl.ds((k0 + k) * D + col, 16)],
                 ws[pl.ds((k0 + k) * D + col, 16)]) for k in range(kk)]

        @plsc.parallel_loop(0, CHUNK, unroll=8)
        def b_body(b):
            boff = b * D
            acc = bias
            for rvec, wvec in taps:
                vals = plsc.load_gather(src, [rvec + boff])
                acc = acc + vals * wvec
            if do_gelu:
                acc = _gelu_exact(acc)
            dst[pl.ds(boff + col, 16)] = acc

        return carry

    lax.fori_loop(0, OCN, oc_body, 0)


def _make_sc_net():
    mesh = plsc.VectorSubcoreMesh(core_axis_name="c", subcore_axis_name="s",
                                  num_cores=NC, num_subcores=NS)

    @functools.partial(
        pl.kernel,
        out_type=jax.ShapeDtypeStruct((B * D,), jnp.float32),
        mesh=mesh,
        compiler_params=pltpu.CompilerParams(
            needs_layout_passes=False, use_tc_tiling_on_sc=False),
        scratch_types=[
            pltpu.VMEM((CHUNK * D,), jnp.float32),
            pltpu.VMEM((CHUNK * D,), jnp.float32),
            pltpu.VMEM((_KTOT * D,), jnp.int32),
            pltpu.VMEM((_KTOT * D,), jnp.float32),
            pltpu.VMEM((3 * D,), jnp.float32),
        ],
    )
    def sc_net(x_hbm, idx_hbm, w_hbm, b_hbm, out_hbm, xs, hs, idxs, ws, bsv):
        wid = lax.axis_index("s") * NC + lax.axis_index("c")
        pltpu.sync_copy(idx_hbm, idxs)
        pltpu.sync_copy(w_hbm, ws)
        pltpu.sync_copy(b_hbm, bsv)
        for p in range(NPASS):
            base = (wid * NPASS + p) * (CHUNK * D)
            pltpu.sync_copy(x_hbm.at[pl.ds(base, CHUNK * D)], xs)
            _sc_layer(xs, hs, idxs, ws, bsv, *_LAYERS[0])
            _sc_layer(hs, xs, idxs, ws, bsv, *_LAYERS[1])
            _sc_layer(xs, hs, idxs, ws, bsv, *_LAYERS[2])
            pltpu.sync_copy(hs, out_hbm.at[pl.ds(base, CHUNK * D)])

    return sc_net


@functools.cache
def _sc_net_cached():
    return _make_sc_net()


def _tc_head_body(h_ref, w_ref, b_ref, o_ref):
    g = _gelu_exact(h_ref[...])
    logits = jnp.dot(g, w_ref[...], preferred_element_type=jnp.float32,
                     precision=lax.Precision.HIGHEST) + b_ref[...]
    m = jnp.max(logits, axis=-1, keepdims=True)
    e = jnp.exp(logits - m)
    o_ref[...] = e / jnp.sum(e, axis=-1, keepdims=True)


def _tc_head(h3, fcw_t, fc_b2):
    blk = 512
    return pl.pallas_call(
        _tc_head_body,
        grid=(B // blk,),
        in_specs=[
            pl.BlockSpec((blk, D), lambda i: (i, 0)),
            pl.BlockSpec((D, NUM_CLASSES), lambda i: (0, 0)),
            pl.BlockSpec((1, NUM_CLASSES), lambda i: (0, 0)),
        ],
        out_specs=pl.BlockSpec((blk, NUM_CLASSES), lambda i: (i, 0)),
        out_shape=jax.ShapeDtypeStruct((B, NUM_CLASSES), jnp.float32),
    )(h3, fcw_t, fc_b2)


def kernel(x, idx1, w1, b1, idx2, w2, b2, idx3, w3, b3, fc_w, fc_b):
    idx_all = jnp.concatenate(
        [idx1.T.astype(jnp.int32), idx2.T.astype(jnp.int32),
         idx3.T.astype(jnp.int32)], axis=0).reshape(-1)
    w_all = jnp.concatenate([w1.T, w2.T, w3.T], axis=0).reshape(-1)
    b_all = jnp.concatenate([b1, b2, b3], axis=0)

    h3 = _sc_net_cached()(x.reshape(-1), idx_all, w_all, b_all).reshape(B, D)
    return _tc_head(h3, fc_w.T, fc_b.reshape(1, NUM_CLASSES))


# speedup vs baseline: 1.0537x; 1.0537x over previous
import functools

import jax
import jax.numpy as jnp
from jax import lax
from jax.experimental import pallas as pl
from jax.experimental.pallas import tpu as pltpu
from jax.experimental.pallas import tpu_sc as plsc

B = 4096
D = 784
NUM_CLASSES = 10

NC = 2
NS = 16
NW = NC * NS
CHUNK = 32
NPASS = B // (NW * CHUNK)
OCN = D // 16

_LAYERS = ((0, 2, 0, True), (2, 4, 1, True), (6, 8, 2, False))
_KTOT = 14


def _gelu_exact(v):
    z = v * 0.7071067811865476
    a = jnp.abs(z)
    t = 1.0 / (1.0 + 0.3275911 * a)
    poly = t * (0.254829592 + t * (-0.284496736 + t * (1.421413741
             + t * (-1.453152027 + t * 1.061405429))))
    erf_a = 1.0 - poly * jnp.exp(-(a * a))
    erf_z = jnp.where(z < 0.0, -erf_a, erf_a)
    return 0.5 * v * (1.0 + erf_z)


def _sc_layer(src, dst, idxs, ws, bsv, k0, kk, brow, do_gelu):
    def oc_body(oc, carry):
        col = oc * 16
        bias = bsv[pl.ds(brow * D + col, 16)]
        taps = [(idxs[pl.ds((k0 + k) * D + col, 16)],
                 ws[pl.ds((k0 + k) * D + col, 16)]) for k in range(kk)]

        @plsc.parallel_loop(0, CHUNK, unroll=4)
        def b_body(b):
            boff = b * D
            acc = bias
            for rvec, wvec in taps:
                vals = plsc.load_gather(src, [rvec + boff])
                acc = acc + vals * wvec
            if do_gelu:
                acc = _gelu_exact(acc)
            dst[pl.ds(boff + col, 16)] = acc

        return carry

    lax.fori_loop(0, OCN, oc_body, 0)


def _make_sc_net():
    mesh = plsc.VectorSubcoreMesh(core_axis_name="c", subcore_axis_name="s",
                                  num_cores=NC, num_subcores=NS)

    @functools.partial(
        pl.kernel,
        out_type=jax.ShapeDtypeStruct((B * D,), jnp.float32),
        mesh=mesh,
        compiler_params=pltpu.CompilerParams(
            needs_layout_passes=False, use_tc_tiling_on_sc=False),
        scratch_types=[
            pltpu.VMEM((CHUNK * D,), jnp.float32),
            pltpu.VMEM((CHUNK * D,), jnp.float32),
            pltpu.VMEM((_KTOT * D,), jnp.int32),
            pltpu.VMEM((_KTOT * D,), jnp.float32),
            pltpu.VMEM((3 * D,), jnp.float32),
        ],
    )
    def sc_net(x_hbm, idx_hbm, w_hbm, b_hbm, out_hbm, xs, hs, idxs, ws, bsv):
        wid = lax.axis_index("s") * NC + lax.axis_index("c")
        pltpu.sync_copy(idx_hbm, idxs)
        pltpu.sync_copy(w_hbm, ws)
        pltpu.sync_copy(b_hbm, bsv)
        for p in range(NPASS):
            base = (wid * NPASS + p) * (CHUNK * D)
            pltpu.sync_copy(x_hbm.at[pl.ds(base, CHUNK * D)], xs)
            _sc_layer(xs, hs, idxs, ws, bsv, *_LAYERS[0])
            _sc_layer(hs, xs, idxs, ws, bsv, *_LAYERS[1])
            _sc_layer(xs, hs, idxs, ws, bsv, *_LAYERS[2])
            pltpu.sync_copy(hs, out_hbm.at[pl.ds(base, CHUNK * D)])

    return sc_net


@functools.cache
def _sc_net_cached():
    return _make_sc_net()


def _tc_head_body(h_ref, w_ref, b_ref, o_ref):
    g = _gelu_exact(h_ref[...])
    logits = jnp.dot(g, w_ref[...], preferred_element_type=jnp.float32,
                     precision=lax.Precision.HIGHEST) + b_ref[...]
    m = jnp.max(logits, axis=-1, keepdims=True)
    e = jnp.exp(logits - m)
    o_ref[...] = e / jnp.sum(e, axis=-1, keepdims=True)


def _tc_head(h3, fcw_t, fc_b2):
    blk = 512
    return pl.pallas_call(
        _tc_head_body,
        grid=(B // blk,),
        in_specs=[
            pl.BlockSpec((blk, D), lambda i: (i, 0)),
            pl.BlockSpec((D, NUM_CLASSES), lambda i: (0, 0)),
            pl.BlockSpec((1, NUM_CLASSES), lambda i: (0, 0)),
        ],
        out_specs=pl.BlockSpec((blk, NUM_CLASSES), lambda i: (i, 0)),
        out_shape=jax.ShapeDtypeStruct((B, NUM_CLASSES), jnp.float32),
    )(h3, fcw_t, fc_b2)


def kernel(x, idx1, w1, b1, idx2, w2, b2, idx3, w3, b3, fc_w, fc_b):
    idx_all = jnp.concatenate(
        [idx1.T.astype(jnp.int32), idx2.T.astype(jnp.int32),
         idx3.T.astype(jnp.int32)], axis=0).reshape(-1)
    w_all = jnp.concatenate([w1.T, w2.T, w3.T], axis=0).reshape(-1)
    b_all = jnp.concatenate([b1, b2, b3], axis=0)

    h3 = _sc_net_cached()(x.reshape(-1), idx_all, w_all, b_all).reshape(B, D)
    return _tc_head(h3, fc_w.T, fc_b.reshape(1, NUM_CLASSES))


# speedup vs baseline: 1.1571x; 1.0981x over previous
import functools

import jax
import jax.numpy as jnp
from jax import lax
from jax.experimental import pallas as pl
from jax.experimental.pallas import tpu as pltpu
from jax.experimental.pallas import tpu_sc as plsc

B = 4096
D = 784
NUM_CLASSES = 10

NC = 2
NS = 16
NW = NC * NS
CHUNK = 32
NPASS = B // (NW * CHUNK)
OCN = D // 16

_LAYERS = ((0, 2, 0, True), (2, 4, 1, True), (6, 8, 2, False))
_KTOT = 14


def _gelu_exact(v):
    z = v * 0.7071067811865476
    a = jnp.abs(z)
    t = 1.0 / (1.0 + 0.3275911 * a)
    poly = t * (0.254829592 + t * (-0.284496736 + t * (1.421413741
             + t * (-1.453152027 + t * 1.061405429))))
    erf_a = 1.0 - poly * jnp.exp(-(a * a))
    erf_z = jnp.where(z < 0.0, -erf_a, erf_a)
    return 0.5 * v * (1.0 + erf_z)


def _gelu_fast(v):
    t = jnp.minimum(v * v, 90.0)
    u = -0.0007098086084286619 * t + 0.07405305138626019
    u = u * t + 1.5949698227920912
    e = jnp.exp(-(u * v))
    return v / (1.0 + e)


def _sc_layer(src, dst, idxs, ws, bsv, k0, kk, brow, do_gelu):
    def oc_body(oc, carry):
        col = oc * 16
        bias = bsv[pl.ds(brow * D + col, 16)]
        taps = [(idxs[pl.ds((k0 + k) * D + col, 16)],
                 ws[pl.ds((k0 + k) * D + col, 16)]) for k in range(kk)]

        @plsc.parallel_loop(0, CHUNK, unroll=4)
        def b_body(b):
            boff = b * D
            acc = bias
            for rvec, wvec in taps:
                vals = plsc.load_gather(src, [rvec + boff])
                acc = acc + vals * wvec
            if do_gelu:
                acc = _gelu_fast(acc)
            dst[pl.ds(boff + col, 16)] = acc

        return carry

    lax.fori_loop(0, OCN, oc_body, 0)


def _make_sc_net():
    mesh = plsc.VectorSubcoreMesh(core_axis_name="c", subcore_axis_name="s",
                                  num_cores=NC, num_subcores=NS)

    @functools.partial(
        pl.kernel,
        out_type=jax.ShapeDtypeStruct((B * D,), jnp.float32),
        mesh=mesh,
        compiler_params=pltpu.CompilerParams(
            needs_layout_passes=False, use_tc_tiling_on_sc=False),
        scratch_types=[
            pltpu.VMEM((CHUNK * D,), jnp.float32),
            pltpu.VMEM((CHUNK * D,), jnp.float32),
            pltpu.VMEM((_KTOT * D,), jnp.int32),
            pltpu.VMEM((_KTOT * D,), jnp.float32),
            pltpu.VMEM((3 * D,), jnp.float32),
        ],
    )
    def sc_net(x_hbm, idx_hbm, w_hbm, b_hbm, out_hbm, xs, hs, idxs, ws, bsv):
        wid = lax.axis_index("s") * NC + lax.axis_index("c")
        pltpu.sync_copy(idx_hbm, idxs)
        pltpu.sync_copy(w_hbm, ws)
        pltpu.sync_copy(b_hbm, bsv)
        for p in range(NPASS):
            row0 = (wid * NPASS + p) * CHUNK
            base = row0 * D
            pltpu.sync_copy(x_hbm.at[pl.ds(base, CHUNK * D)], xs)
            _sc_layer(xs, hs, idxs, ws, bsv, *_LAYERS[0])
            _sc_layer(hs, xs, idxs, ws, bsv, *_LAYERS[1])
            _sc_layer(xs, hs, idxs, ws, bsv, *_LAYERS[2])
            pltpu.sync_copy(hs, out_hbm.at[pl.ds(base, CHUNK * D)])

    return sc_net


@functools.cache
def _sc_net_cached():
    return _make_sc_net()


def _tc_head_body(h_ref, w_ref, b_ref, o_ref):
    g = _gelu_exact(h_ref[...])
    logits = jnp.dot(g, w_ref[...], preferred_element_type=jnp.float32,
                     precision=lax.Precision.HIGHEST) + b_ref[...]
    m = jnp.max(logits, axis=-1, keepdims=True)
    e = jnp.exp(logits - m)
    o_ref[...] = e / jnp.sum(e, axis=-1, keepdims=True)


def _tc_head(h3, fcw_t, fc_b2):
    blk = 512
    return pl.pallas_call(
        _tc_head_body,
        grid=(B // blk,),
        in_specs=[
            pl.BlockSpec((blk, D), lambda i: (i, 0)),
            pl.BlockSpec((D, NUM_CLASSES), lambda i: (0, 0)),
            pl.BlockSpec((1, NUM_CLASSES), lambda i: (0, 0)),
        ],
        out_specs=pl.BlockSpec((blk, NUM_CLASSES), lambda i: (i, 0)),
        out_shape=jax.ShapeDtypeStruct((B, NUM_CLASSES), jnp.float32),
    )(h3, fcw_t, fc_b2)


def kernel(x, idx1, w1, b1, idx2, w2, b2, idx3, w3, b3, fc_w, fc_b):
    idx_all = jnp.concatenate(
        [idx1.T.astype(jnp.int32), idx2.T.astype(jnp.int32),
         idx3.T.astype(jnp.int32)], axis=0).reshape(-1)
    w_all = jnp.concatenate([w1.T, w2.T, w3.T], axis=0).reshape(-1)
    b_all = jnp.concatenate([b1, b2, b3], axis=0)

    h3 = _sc_net_cached()(x.reshape(-1), idx_all, w_all, b_all).reshape(B, D)
    return _tc_head(h3, fc_w.T, fc_b.reshape(1, NUM_CLASSES))


# speedup vs baseline: 1.1765x; 1.0168x over previous
import functools

import jax
import jax.numpy as jnp
from jax import lax
from jax.experimental import pallas as pl
from jax.experimental.pallas import tpu as pltpu
from jax.experimental.pallas import tpu_sc as plsc

B = 4096
D = 784
NUM_CLASSES = 10

NC = 2
NS = 16
NW = NC * NS
CHUNK = 32
NPASS = B // (NW * CHUNK)
OCN = D // 16

_LAYERS = ((0, 2, 0, True), (2, 4, 1, True), (6, 8, 2, False))
_KTOT = 14


def _gelu_exact(v):
    z = v * 0.7071067811865476
    a = jnp.abs(z)
    t = 1.0 / (1.0 + 0.3275911 * a)
    poly = t * (0.254829592 + t * (-0.284496736 + t * (1.421413741
             + t * (-1.453152027 + t * 1.061405429))))
    erf_a = 1.0 - poly * jnp.exp(-(a * a))
    erf_z = jnp.where(z < 0.0, -erf_a, erf_a)
    return 0.5 * v * (1.0 + erf_z)


def _gelu_fast(v):
    t = jnp.minimum(v * v, 90.0)
    u = -0.0007098086084286619 * t + 0.07405305138626019
    u = u * t + 1.5949698227920912
    e = jnp.exp(-(u * v))
    return v / (1.0 + e)


def _sc_layer(src, dst, idxs, ws, bsv, k0, kk, brow, do_gelu):
    def oc_body(oc, carry):
        col = oc * 16
        bias = bsv[pl.ds(brow * D + col, 16)]
        taps = [(idxs[pl.ds((k0 + k) * D + col, 16)],
                 ws[pl.ds((k0 + k) * D + col, 16)]) for k in range(kk)]

        @plsc.parallel_loop(0, CHUNK, unroll=4)
        def b_body(b):
            boff = b * D
            acc = bias
            for rvec, wvec in taps:
                vals = plsc.load_gather(src, [rvec + boff])
                acc = acc + vals * wvec
            if do_gelu:
                acc = _gelu_fast(acc)
            dst[pl.ds(boff + col, 16)] = acc

        return carry

    lax.fori_loop(0, OCN, oc_body, 0)


def _make_sc_net():
    mesh = plsc.VectorSubcoreMesh(core_axis_name="c", subcore_axis_name="s",
                                  num_cores=NC, num_subcores=NS)

    @functools.partial(
        pl.kernel,
        out_type=jax.ShapeDtypeStruct((B, D), jnp.float32),
        mesh=mesh,
        compiler_params=pltpu.CompilerParams(
            needs_layout_passes=False, use_tc_tiling_on_sc=False),
        scratch_types=[
            pltpu.VMEM((CHUNK * D,), jnp.float32),
            pltpu.VMEM((CHUNK * D,), jnp.float32),
            pltpu.VMEM((_KTOT * D,), jnp.int32),
            pltpu.VMEM((_KTOT * D,), jnp.float32),
            pltpu.VMEM((3 * D,), jnp.float32),
            pltpu.SemaphoreType.DMA,
        ],
    )
    def sc_net(x_hbm, idx_hbm, w_hbm, b_hbm, out_hbm, xs, hs, idxs, ws, bsv, sem):
        wid = lax.axis_index("s") * NC + lax.axis_index("c")
        pltpu.sync_copy(idx_hbm, idxs)
        pltpu.sync_copy(w_hbm, ws)
        pltpu.sync_copy(b_hbm, bsv)
        for p in range(NPASS):
            row0 = (wid * NPASS + p) * CHUNK
            loads = [pltpu.async_copy(x_hbm.at[row0 + b],
                                      xs.at[pl.ds(b * D, D)], sem)
                     for b in range(CHUNK)]
            for cp in loads:
                cp.wait()
            _sc_layer(xs, hs, idxs, ws, bsv, *_LAYERS[0])
            _sc_layer(hs, xs, idxs, ws, bsv, *_LAYERS[1])
            _sc_layer(xs, hs, idxs, ws, bsv, *_LAYERS[2])
            stores = [pltpu.async_copy(hs.at[pl.ds(b * D, D)],
                                       out_hbm.at[row0 + b], sem)
                      for b in range(CHUNK)]
            for cp in stores:
                cp.wait()

    return sc_net


@functools.cache
def _sc_net_cached():
    return _make_sc_net()


def _tc_head_body(h_ref, w_ref, b_ref, o_ref):
    g = _gelu_fast(h_ref[...])
    logits = jnp.dot(g, w_ref[...], preferred_element_type=jnp.float32,
                     precision=lax.Precision.HIGHEST) + b_ref[...]
    m = jnp.max(logits, axis=-1, keepdims=True)
    e = jnp.exp(logits - m)
    o_ref[...] = e / jnp.sum(e, axis=-1, keepdims=True)


def _tc_head(h3, fcw_t, fc_b2):
    blk = 1024
    return pl.pallas_call(
        _tc_head_body,
        grid=(B // blk,),
        in_specs=[
            pl.BlockSpec((blk, D), lambda i: (i, 0)),
            pl.BlockSpec((D, NUM_CLASSES), lambda i: (0, 0)),
            pl.BlockSpec((1, NUM_CLASSES), lambda i: (0, 0)),
        ],
        out_specs=pl.BlockSpec((blk, NUM_CLASSES), lambda i: (i, 0)),
        out_shape=jax.ShapeDtypeStruct((B, NUM_CLASSES), jnp.float32),
    )(h3, fcw_t, fc_b2)


def kernel(x, idx1, w1, b1, idx2, w2, b2, idx3, w3, b3, fc_w, fc_b):
    idx_all = jnp.concatenate(
        [idx1.T.astype(jnp.int32), idx2.T.astype(jnp.int32),
         idx3.T.astype(jnp.int32)], axis=0).reshape(-1)
    w_all = jnp.concatenate([w1.T, w2.T, w3.T], axis=0).reshape(-1)
    b_all = jnp.concatenate([b1, b2, b3], axis=0)

    h3 = _sc_net_cached()(x, idx_all, w_all, b_all)
    return _tc_head(h3, fc_w.T, fc_b.reshape(1, NUM_CLASSES))


# speedup vs baseline: 1.1782x; 1.0015x over previous
import functools

import jax
import jax.numpy as jnp
from jax import lax
from jax.experimental import pallas as pl
from jax.experimental.pallas import tpu as pltpu
from jax.experimental.pallas import tpu_sc as plsc

B = 4096
D = 784
NUM_CLASSES = 10

NC = 2
NS = 16
NW = NC * NS
CHUNK = 32
NPASS = B // (NW * CHUNK)
OCN = D // 16

_LAYERS = ((0, 2, 0, True), (2, 4, 1, True), (6, 8, 2, False))
_KTOT = 14


def _gelu_exact(v):
    z = v * 0.7071067811865476
    a = jnp.abs(z)
    t = 1.0 / (1.0 + 0.3275911 * a)
    poly = t * (0.254829592 + t * (-0.284496736 + t * (1.421413741
             + t * (-1.453152027 + t * 1.061405429))))
    erf_a = 1.0 - poly * jnp.exp(-(a * a))
    erf_z = jnp.where(z < 0.0, -erf_a, erf_a)
    return 0.5 * v * (1.0 + erf_z)


def _gelu_fast(v):
    t = jnp.minimum(v * v, 90.0)
    u = -0.0007098086084286619 * t + 0.07405305138626019
    u = u * t + 1.5949698227920912
    e = jnp.exp(-(u * v))
    return v / (1.0 + e)


def _sc_layer(src, dst, idxs, ws, bsv, k0, kk, brow, do_gelu):
    @plsc.parallel_loop(0, OCN)
    def oc_body(oc):
        col = oc * 16
        bias = bsv[pl.ds(brow * D + col, 16)]
        taps = [(idxs[pl.ds((k0 + k) * D + col, 16)],
                 ws[pl.ds((k0 + k) * D + col, 16)]) for k in range(kk)]

        @plsc.parallel_loop(0, CHUNK, unroll=4)
        def b_body(b):
            boff = b * D
            acc = bias
            for rvec, wvec in taps:
                vals = plsc.load_gather(src, [rvec + boff])
                acc = acc + vals * wvec
            if do_gelu:
                acc = _gelu_fast(acc)
            dst[pl.ds(boff + col, 16)] = acc


def _make_sc_net():
    mesh = plsc.VectorSubcoreMesh(core_axis_name="c", subcore_axis_name="s",
                                  num_cores=NC, num_subcores=NS)

    @functools.partial(
        pl.kernel,
        out_type=jax.ShapeDtypeStruct((B, D), jnp.float32),
        mesh=mesh,
        compiler_params=pltpu.CompilerParams(
            needs_layout_passes=False, use_tc_tiling_on_sc=False),
        scratch_types=[
            pltpu.VMEM((CHUNK * D,), jnp.float32),
            pltpu.VMEM((CHUNK * D,), jnp.float32),
            pltpu.VMEM((_KTOT * D,), jnp.int32),
            pltpu.VMEM((_KTOT * D,), jnp.float32),
            pltpu.VMEM((3 * D,), jnp.float32),
            pltpu.SemaphoreType.DMA,
        ],
    )
    def sc_net(x_hbm, idx_hbm, w_hbm, b_hbm, out_hbm, xs, hs, idxs, ws, bsv, sem):
        wid = lax.axis_index("s") * NC + lax.axis_index("c")
        pltpu.sync_copy(idx_hbm, idxs)
        pltpu.sync_copy(w_hbm, ws)
        pltpu.sync_copy(b_hbm, bsv)
        for p in range(NPASS):
            row0 = (wid * NPASS + p) * CHUNK
            loads = [pltpu.async_copy(x_hbm.at[row0 + b],
                                      xs.at[pl.ds(b * D, D)], sem)
                     for b in range(CHUNK)]
            for cp in loads:
                cp.wait()
            _sc_layer(xs, hs, idxs, ws, bsv, *_LAYERS[0])
            _sc_layer(hs, xs, idxs, ws, bsv, *_LAYERS[1])
            _sc_layer(xs, hs, idxs, ws, bsv, *_LAYERS[2])
            stores = [pltpu.async_copy(hs.at[pl.ds(b * D, D)],
                                       out_hbm.at[row0 + b], sem)
                      for b in range(CHUNK)]
            for cp in stores:
                cp.wait()

    return sc_net


@functools.cache
def _sc_net_cached():
    return _make_sc_net()


def _tc_head_body(h_ref, w_ref, b_ref, o_ref):
    g = _gelu_fast(h_ref[...])
    logits = jnp.dot(g, w_ref[...], preferred_element_type=jnp.float32,
                     precision=lax.Precision.HIGHEST) + b_ref[...]
    m = jnp.max(logits, axis=-1, keepdims=True)
    e = jnp.exp(logits - m)
    o_ref[...] = e / jnp.sum(e, axis=-1, keepdims=True)


def _tc_head(h3, fcw_t, fc_b2):
    blk = 1024
    return pl.pallas_call(
        _tc_head_body,
        grid=(B // blk,),
        in_specs=[
            pl.BlockSpec((blk, D), lambda i: (i, 0)),
            pl.BlockSpec((D, NUM_CLASSES), lambda i: (0, 0)),
            pl.BlockSpec((1, NUM_CLASSES), lambda i: (0, 0)),
        ],
        out_specs=pl.BlockSpec((blk, NUM_CLASSES), lambda i: (i, 0)),
        out_shape=jax.ShapeDtypeStruct((B, NUM_CLASSES), jnp.float32),
    )(h3, fcw_t, fc_b2)


def kernel(x, idx1, w1, b1, idx2, w2, b2, idx3, w3, b3, fc_w, fc_b):
    idx_all = jnp.concatenate(
        [idx1.T.astype(jnp.int32), idx2.T.astype(jnp.int32),
         idx3.T.astype(jnp.int32)], axis=0).reshape(-1)
    w_all = jnp.concatenate([w1.T, w2.T, w3.T], axis=0).reshape(-1)
    b_all = jnp.concatenate([b1, b2, b3], axis=0)

    h3 = _sc_net_cached()(x, idx_all, w_all, b_all)
    return _tc_head(h3, fc_w.T, fc_b.reshape(1, NUM_CLASSES))


# speedup vs baseline: 1.2264x; 1.0409x over previous
import functools

import jax
import jax.numpy as jnp
from jax import lax
from jax.experimental import pallas as pl
from jax.experimental.pallas import tpu as pltpu
from jax.experimental.pallas import tpu_sc as plsc

B = 4096
D = 784
NUM_CLASSES = 10

NC = 2
NS = 16
NW = NC * NS
CHUNK = 64
NPASS = B // (NW * CHUNK)
OCN = D // 16

_LAYERS = ((0, 2, 0, True), (2, 4, 1, True), (6, 8, 2, False))
_KTOT = 14


def _gelu_exact(v):
    z = v * 0.7071067811865476
    a = jnp.abs(z)
    t = 1.0 / (1.0 + 0.3275911 * a)
    poly = t * (0.254829592 + t * (-0.284496736 + t * (1.421413741
             + t * (-1.453152027 + t * 1.061405429))))
    erf_a = 1.0 - poly * jnp.exp(-(a * a))
    erf_z = jnp.where(z < 0.0, -erf_a, erf_a)
    return 0.5 * v * (1.0 + erf_z)


def _gelu_fast(v):
    t = jnp.minimum(v * v, 90.0)
    u = -0.0007098086084286619 * t + 0.07405305138626019
    u = u * t + 1.5949698227920912
    e = jnp.exp(-(u * v))
    return v / (1.0 + e)


def _sc_layer(src, dst, idxs, ws, bsv, k0, kk, brow, do_gelu):
    @plsc.parallel_loop(0, OCN)
    def oc_body(oc):
        col = oc * 16
        bias = bsv[pl.ds(brow * D + col, 16)]
        taps = [(idxs[pl.ds((k0 + k) * D + col, 16)],
                 ws[pl.ds((k0 + k) * D + col, 16)]) for k in range(kk)]

        @plsc.parallel_loop(0, CHUNK, unroll=4)
        def b_body(b):
            boff = b * D
            acc = bias
            for rvec, wvec in taps:
                vals = plsc.load_gather(src, [rvec + boff])
                acc = acc + vals * wvec
            if do_gelu:
                acc = _gelu_fast(acc)
            dst[pl.ds(boff + col, 16)] = acc


def _make_sc_net():
    mesh = plsc.VectorSubcoreMesh(core_axis_name="c", subcore_axis_name="s",
                                  num_cores=NC, num_subcores=NS)

    @functools.partial(
        pl.kernel,
        out_type=jax.ShapeDtypeStruct((B, D), jnp.float32),
        mesh=mesh,
        compiler_params=pltpu.CompilerParams(
            needs_layout_passes=False, use_tc_tiling_on_sc=False),
        scratch_types=[
            pltpu.VMEM((CHUNK * D,), jnp.float32),
            pltpu.VMEM((CHUNK * D,), jnp.float32),
            pltpu.VMEM((_KTOT * D,), jnp.int32),
            pltpu.VMEM((_KTOT * D,), jnp.float32),
            pltpu.VMEM((3 * D,), jnp.float32),
            pltpu.SemaphoreType.DMA,
        ],
    )
    def sc_net(x_hbm, idx_hbm, w_hbm, b_hbm, out_hbm, xs, hs, idxs, ws, bsv, sem):
        wid = lax.axis_index("s") * NC + lax.axis_index("c")
        pltpu.sync_copy(idx_hbm, idxs)
        pltpu.sync_copy(w_hbm, ws)
        pltpu.sync_copy(b_hbm, bsv)
        for p in range(NPASS):
            row0 = (wid * NPASS + p) * CHUNK
            loads = [pltpu.async_copy(x_hbm.at[row0 + b],
                                      xs.at[pl.ds(b * D, D)], sem)
                     for b in range(CHUNK)]
            for cp in loads:
                cp.wait()
            _sc_layer(xs, hs, idxs, ws, bsv, *_LAYERS[0])
            _sc_layer(hs, xs, idxs, ws, bsv, *_LAYERS[1])
            _sc_layer(xs, hs, idxs, ws, bsv, *_LAYERS[2])
            stores = [pltpu.async_copy(hs.at[pl.ds(b * D, D)],
                                       out_hbm.at[row0 + b], sem)
                      for b in range(CHUNK)]
            for cp in stores:
                cp.wait()

    return sc_net


@functools.cache
def _sc_net_cached():
    return _make_sc_net()


def _tc_head_body(h_ref, w_ref, b_ref, o_ref):
    g = _gelu_fast(h_ref[...])
    logits = jnp.dot(g, w_ref[...], preferred_element_type=jnp.float32,
                     precision=lax.Precision.HIGHEST) + b_ref[...]
    m = jnp.max(logits, axis=-1, keepdims=True)
    e = jnp.exp(logits - m)
    o_ref[...] = e / jnp.sum(e, axis=-1, keepdims=True)


def _tc_head(h3, fcw_t, fc_b2):
    blk = 1024
    return pl.pallas_call(
        _tc_head_body,
        grid=(B // blk,),
        in_specs=[
            pl.BlockSpec((blk, D), lambda i: (i, 0)),
            pl.BlockSpec((D, NUM_CLASSES), lambda i: (0, 0)),
            pl.BlockSpec((1, NUM_CLASSES), lambda i: (0, 0)),
        ],
        out_specs=pl.BlockSpec((blk, NUM_CLASSES), lambda i: (i, 0)),
        out_shape=jax.ShapeDtypeStruct((B, NUM_CLASSES), jnp.float32),
    )(h3, fcw_t, fc_b2)


def kernel(x, idx1, w1, b1, idx2, w2, b2, idx3, w3, b3, fc_w, fc_b):
    idx_all = jnp.concatenate(
        [idx1.T.astype(jnp.int32), idx2.T.astype(jnp.int32),
         idx3.T.astype(jnp.int32)], axis=0).reshape(-1)
    w_all = jnp.concatenate([w1.T, w2.T, w3.T], axis=0).reshape(-1)
    b_all = jnp.concatenate([b1, b2, b3], axis=0)

    h3 = _sc_net_cached()(x, idx_all, w_all, b_all)
    return _tc_head(h3, fc_w.T, fc_b.reshape(1, NUM_CLASSES))
